# baseline (speedup 1.0000x reference)
import functools

import jax
import jax.numpy as jnp
from jax import lax
from jax.experimental import pallas as pl
from jax.experimental.pallas import tpu as pltpu
from jax.experimental.pallas import tpu_sc as plsc

_SC_PARAMS = pltpu.CompilerParams(
    use_tc_tiling_on_sc=True, needs_layout_passes=False
)


def _relayout_kernel(V, D, NC, NW):
    U = 256
    R = U // 2
    D2 = 2 * D
    n_units = V // U
    tail = V - n_units * U
    iters = (n_units + NW - 1) // NW
    iters += iters % 2
    mesh = plsc.VectorSubcoreMesh(core_axis_name="c", subcore_axis_name="s")

    @functools.partial(
        pl.kernel,
        mesh=mesh,
        out_type=jax.ShapeDtypeStruct((V // 2, D2), jnp.float32),
        scratch_types=[
            pltpu.VMEM((2, D, U), jnp.float32),
            pltpu.VMEM((2, R, D2), jnp.float32),
            pltpu.SemaphoreType.DMA,
            pltpu.SemaphoreType.DMA,
        ],
        compiler_params=_SC_PARAMS,
    )
    def k(tableT_hbm, tail_hbm, packed_hbm, nat_v, pk_v, sin, sout):
        wid = lax.axis_index("s") * NC + lax.axis_index("c")

        def unit_id(i):
            return wid + i * NW

        def lane0(u):
            return pl.multiple_of(u * U, U)

        def stage(i, b):
            u = unit_id(i)

            @pl.when(u < n_units)
            def _():
                pltpu.async_copy(
                    tableT_hbm.at[:, pl.ds(lane0(u), U)], nat_v.at[b], sin
                )

        def drain_stage(i, b):
            @pl.when(unit_id(i) < n_units)
            def _():
                pltpu.make_async_copy(
                    tableT_hbm.at[:, pl.ds(0, U)], nat_v.at[b], sin
                ).wait()

        def write(i, b):
            u = unit_id(i)

            @pl.when(u < n_units)
            def _():
                row0 = pl.multiple_of(u * R, R)
                pltpu.async_copy(
                    pk_v.at[b], packed_hbm.at[pl.ds(row0, R)], sout
                )

        def drain_write(i, b):
            @pl.when(jnp.logical_and(i >= 0, unit_id(i) < n_units))
            def _():
                pltpu.make_async_copy(
                    pk_v.at[b], packed_hbm.at[pl.ds(0, R)], sout
                ).wait()

        def shuffle(b):
            nat = nat_v.at[b]
            pk = pk_v.at[b]
            lanes = lax.iota(jnp.int32, 16)

            def row(r, _):
                e = jnp.full((16,), 1, jnp.int32) * (2 * r)
                o = e + 1
                for g in range(D // 16):
                    cols = lanes + (g * 16)
                    pk[r, pl.ds(g * 16, 16)] = plsc.load_gather(nat, [cols, e])
                    pk[r, pl.ds(D + g * 16, 16)] = plsc.load_gather(
                        nat, [cols, o]
                    )
                return 0

            lax.fori_loop(0, R, row, 0)

        stage(0, 0)
        def body(i2_, _):
            i2 = i2_ * 2
            for b in range(2):
                i = i2 + b
                u = unit_id(i)
                drain_stage(i, b)
                stage(i + 1, 1 - b)
                drain_write(i - 2, b)

                @pl.when(u < n_units)
                def _():
                    shuffle(b)

                write(i, b)
            return 0

        lax.fori_loop(0, iters // 2, body, 0)
        drain_stage(iters, iters % 2)
        drain_write(iters - 2, 0)
        drain_write(iters - 1, 1)

        if tail:
            @pl.when(wid == 0)
            def _():
                pltpu.sync_copy(tail_hbm, pk_v.at[0, pl.ds(0, tail // 2)])
                pltpu.sync_copy(
                    pk_v.at[0, pl.ds(0, tail // 2)],
                    packed_hbm.at[pl.ds(n_units * R, tail // 2)],
                )

    return k


def _gather_kernel(B, D, b_per_w, CH, NC):
    n_ch = b_per_w // CH
    D2 = 2 * D
    mesh = plsc.VectorSubcoreMesh(core_axis_name="c", subcore_axis_name="s")

    @functools.partial(
        pl.kernel,
        mesh=mesh,
        out_type=jax.ShapeDtypeStruct((D, B), jnp.float32),
        scratch_types=[
            pltpu.VMEM((b_per_w,), jnp.int32),
            pltpu.VMEM((b_per_w,), jnp.int32),
            pltpu.VMEM((b_per_w, D2), jnp.float32),
            pltpu.VMEM((D, b_per_w), jnp.float32),
            pltpu.SemaphoreType.DMA,
        ],
        compiler_params=_SC_PARAMS,
    )
    def k(packed_hbm, idx_hbm, outT_hbm, idx_v, blk_v, rows_v, cols_v, sem):
        wid = lax.axis_index("s") * NC + lax.axis_index("c")
        base = wid * b_per_w
        pltpu.sync_copy(idx_hbm.at[pl.ds(base, b_per_w)], idx_v)
        for v in range(b_per_w // 16):
            iv = idx_v[pl.ds(v * 16, 16)]
            blk_v[pl.ds(v * 16, 16)] = iv >> 1
        copies = [
            pltpu.async_copy(
                packed_hbm.at[blk_v.at[pl.ds(c * CH, CH)]],
                rows_v.at[pl.ds(c * CH, CH)],
                sem,
            )
            for c in range(n_ch)
        ]
        for cp in copies:
            cp.wait()
        lanes = lax.iota(jnp.int32, 16)
        for g in range(b_per_w // 16):
            iv = idx_v[pl.ds(g * 16, 16)]
            colbase = (iv & 1) * D
            rowids = lanes + (g * 16)
            for c in range(D):
                vals = plsc.load_gather(rows_v, [rowids, colbase + c])
                cols_v[c, pl.ds(g * 16, 16)] = vals
        pltpu.sync_copy(cols_v, outT_hbm.at[:, pl.ds(base, b_per_w)])

    return k


def kernel(input, index):
    V, D = input.shape
    B = index.shape[0]
    idx32 = index.astype(jnp.int32)
    tableT = input.T

    info = plsc.get_sparse_core_info()
    NC, NS = info.num_cores, info.num_subcores
    NW = NC * NS
    b_per_w = B // NW

    tail = V - (V // 256) * 256
    tail_packed = input[V - tail :, :].reshape(tail // 2, 2 * D)
    packed = _relayout_kernel(V, D, NC, NW)(tableT, tail_packed)
    outT = _gather_kernel(B, D, b_per_w, 128, NC)(packed, idx32)
    return outT.T


# speedup vs baseline: 3.0928x; 3.0928x over previous
import functools

import jax
import jax.numpy as jnp
from jax import lax
from jax.experimental import pallas as pl
from jax.experimental.pallas import tpu as pltpu
from jax.experimental.pallas import tpu_sc as plsc

_SC_PARAMS = pltpu.CompilerParams(
    use_tc_tiling_on_sc=True, needs_layout_passes=False
)
_G = 16


def _relayout_kernel(V, D, NC, NW):
    U = 256
    Q = U // _G
    SW = D * _G
    n_units = V // U
    tail = V - n_units * U
    n_slabs = (V + _G - 1) // _G
    iters = (n_units + NW - 1) // NW
    iters += iters % 2
    mesh = plsc.VectorSubcoreMesh(core_axis_name="c", subcore_axis_name="s")

    @functools.partial(
        pl.kernel,
        mesh=mesh,
        out_type=jax.ShapeDtypeStruct((n_slabs, SW), jnp.float32),
        scratch_types=[
            pltpu.VMEM((2, D, U), jnp.float32),
            pltpu.VMEM((2, Q, SW), jnp.float32),
            pltpu.SemaphoreType.DMA,
            pltpu.SemaphoreType.DMA,
        ],
        compiler_params=_SC_PARAMS,
    )
    def k(tableT_hbm, tail_hbm, slabs_hbm, nat_v, pk_v, sin, sout):
        wid = lax.axis_index("s") * NC + lax.axis_index("c")

        def unit_id(i):
            return wid + i * NW

        def stage(i, b):
            u = unit_id(i)

            @pl.when(u < n_units)
            def _():
                lane0 = pl.multiple_of(u * U, U)
                pltpu.async_copy(
                    tableT_hbm.at[:, pl.ds(lane0, U)], nat_v.at[b], sin
                )

        def drain_stage(i, b):
            @pl.when(unit_id(i) < n_units)
            def _():
                pltpu.make_async_copy(
                    tableT_hbm.at[:, pl.ds(0, U)], nat_v.at[b], sin
                ).wait()

        def write(i, b):
            u = unit_id(i)

            @pl.when(u < n_units)
            def _():
                row0 = pl.multiple_of(u * Q, Q)
                pltpu.async_copy(
                    pk_v.at[b], slabs_hbm.at[pl.ds(row0, Q)], sout
                )

        def drain_write(i, b):
            @pl.when(jnp.logical_and(i >= 0, unit_id(i) < n_units))
            def _():
                pltpu.make_async_copy(
                    pk_v.at[b], slabs_hbm.at[pl.ds(0, Q)], sout
                ).wait()

        def shuffle(b):
            nat = nat_v.at[b]
            pk = pk_v.at[b]

            def row(q, _):
                l0 = q * _G
                for c in range(D):
                    pk[q, pl.ds(c * _G, _G)] = nat[c, pl.ds(l0, _G)]
                return 0

            lax.fori_loop(0, Q, row, 0)

        stage(0, 0)
        def body(i2_, _):
            i2 = i2_ * 2
            for b in range(2):
                i = i2 + b
                u = unit_id(i)
                drain_stage(i, b)
                stage(i + 1, 1 - b)
                drain_write(i - 2, b)

                @pl.when(u < n_units)
                def _():
                    shuffle(b)

                write(i, b)
            return 0

        lax.fori_loop(0, iters // 2, body, 0)
        drain_stage(iters, iters % 2)
        drain_write(iters - 2, 0)
        drain_write(iters - 1, 1)

        if tail:
            @pl.when(wid == 0)
            def _():
                nt = tail // _G
                pltpu.sync_copy(tail_hbm, pk_v.at[0, pl.ds(0, nt)])
                pltpu.sync_copy(
                    pk_v.at[0, pl.ds(0, nt)],
                    slabs_hbm.at[pl.ds(n_units * Q, nt)],
                )

    return k


def _gather_kernel(B, D, b_per_w, CH, NC):
    n_ch = b_per_w // CH
    SW = D * _G
    mesh = plsc.VectorSubcoreMesh(core_axis_name="c", subcore_axis_name="s")

    @functools.partial(
        pl.kernel,
        mesh=mesh,
        out_type=jax.ShapeDtypeStruct((D, B), jnp.float32),
        scratch_types=[
            pltpu.VMEM((b_per_w,), jnp.int32),
            pltpu.VMEM((b_per_w,), jnp.int32),
            pltpu.VMEM((2, CH, SW), jnp.float32),
            pltpu.VMEM((D, b_per_w), jnp.float32),
            pltpu.SemaphoreType.DMA,
        ],
        compiler_params=_SC_PARAMS,
    )
    def k(slabs_hbm, idx_hbm, outT_hbm, idx_v, blk_v, rows_v, cols_v, sem):
        wid = lax.axis_index("s") * NC + lax.axis_index("c")
        base = wid * b_per_w
        pltpu.sync_copy(idx_hbm.at[pl.ds(base, b_per_w)], idx_v)
        for v in range(b_per_w // 16):
            iv = idx_v[pl.ds(v * 16, 16)]
            blk_v[pl.ds(v * 16, 16)] = iv >> 4

        def start(c, b):
            @pl.when(c < n_ch)
            def _():
                i0 = pl.multiple_of(c * CH, CH)
                pltpu.async_copy(
                    slabs_hbm.at[blk_v.at[pl.ds(i0, CH)]],
                    rows_v.at[b],
                    sem,
                )

        def drain():
            pltpu.make_async_copy(
                slabs_hbm.at[blk_v.at[pl.ds(0, CH)]], rows_v.at[0], sem
            ).wait()

        lanes = lax.iota(jnp.int32, 16)
        start(0, 0)

        def body(c2, _):
            for b in range(2):
                c = c2 * 2 + b
                drain()
                start(c + 1, 1 - b)
                buf = rows_v.at[b]
                for g in range(CH // 16):
                    j = c * CH + g * 16
                    iv = idx_v[pl.ds(j, 16)]
                    sub = iv & (_G - 1)
                    rowids = lanes + (g * 16)
                    for col in range(D):
                        vals = plsc.load_gather(
                            buf, [rowids, sub + (col * _G)]
                        )
                        cols_v[col, pl.ds(j, 16)] = vals
            return 0

        lax.fori_loop(0, n_ch // 2, body, 0)
        pltpu.sync_copy(cols_v, outT_hbm.at[:, pl.ds(base, b_per_w)])

    return k


def kernel(input, index):
    V, D = input.shape
    B = index.shape[0]
    idx32 = index.astype(jnp.int32)
    tableT = input.T
    tail = V - (V // 256) * 256
    tail_packed = (
        input[V - tail :, :].reshape(tail // _G, _G, D)
        .transpose(0, 2, 1)
        .reshape(tail // _G, D * _G)
    )

    info = plsc.get_sparse_core_info()
    NC, NS = info.num_cores, info.num_subcores
    NW = NC * NS
    b_per_w = B // NW

    slabs = _relayout_kernel(V, D, NC, NW)(tableT, tail_packed)
    outT = _gather_kernel(B, D, b_per_w, 32, NC)(slabs, idx32)
    return outT.T


# speedup vs baseline: 3.5526x; 1.1487x over previous
import functools

import jax
import jax.numpy as jnp
from jax import lax
from jax.experimental import pallas as pl
from jax.experimental.pallas import tpu as pltpu
from jax.experimental.pallas import tpu_sc as plsc

_SC_PARAMS = pltpu.CompilerParams(
    use_tc_tiling_on_sc=True, needs_layout_passes=False
)
_G = 16


def _relayout_kernel(V, D, NC, NW):
    U = 256
    Q = U // _G
    SW = D * _G
    n_units = V // U
    tail = V - n_units * U
    n_slabs = (V + _G - 1) // _G
    iters = (n_units + NW - 1) // NW
    iters += iters % 2
    mesh = plsc.VectorSubcoreMesh(core_axis_name="c", subcore_axis_name="s")

    @functools.partial(
        pl.kernel,
        mesh=mesh,
        out_type=jax.ShapeDtypeStruct((n_slabs, SW), jnp.float32),
        scratch_types=[
            pltpu.VMEM((2, D, U), jnp.float32),
            pltpu.VMEM((2, Q, SW), jnp.float32),
            pltpu.SemaphoreType.DMA,
            pltpu.SemaphoreType.DMA,
        ],
        compiler_params=_SC_PARAMS,
    )
    def k(tableT_hbm, tail_hbm, slabs_hbm, nat_v, pk_v, sin, sout):
        wid = lax.axis_index("s") * NC + lax.axis_index("c")

        def unit_id(i):
            return wid + i * NW

        def stage(i, b):
            u = unit_id(i)

            @pl.when(u < n_units)
            def _():
                lane0 = pl.multiple_of(u * U, U)
                pltpu.async_copy(
                    tableT_hbm.at[:, pl.ds(lane0, U)], nat_v.at[b], sin
                )

        def drain_stage(i, b):
            @pl.when(unit_id(i) < n_units)
            def _():
                pltpu.make_async_copy(
                    tableT_hbm.at[:, pl.ds(0, U)], nat_v.at[b], sin
                ).wait()

        def write(i, b):
            u = unit_id(i)

            @pl.when(u < n_units)
            def _():
                row0 = pl.multiple_of(u * Q, Q)
                pltpu.async_copy(
                    pk_v.at[b], slabs_hbm.at[pl.ds(row0, Q)], sout
                )

        def drain_write(i, b):
            @pl.when(jnp.logical_and(i >= 0, unit_id(i) < n_units))
            def _():
                pltpu.make_async_copy(
                    pk_v.at[b], slabs_hbm.at[pl.ds(0, Q)], sout
                ).wait()

        def shuffle(b):
            nat = nat_v.at[b]
            pk = pk_v.at[b]

            @plsc.parallel_loop(0, Q, 1, unroll=4)
            def row(q):
                l0 = q * _G
                for c in range(D):
                    pk[q, pl.ds(c * _G, _G)] = nat[c, pl.ds(l0, _G)]

        stage(0, 0)
        def body(i2_, _):
            i2 = i2_ * 2
            for b in range(2):
                i = i2 + b
                u = unit_id(i)
                drain_stage(i, b)
                stage(i + 1, 1 - b)
                drain_write(i - 2, b)

                @pl.when(u < n_units)
                def _():
                    shuffle(b)

                write(i, b)
            return 0

        lax.fori_loop(0, iters // 2, body, 0)
        drain_stage(iters, iters % 2)
        drain_write(iters - 2, 0)
        drain_write(iters - 1, 1)

        if tail:
            @pl.when(wid == 0)
            def _():
                nt = tail // _G
                pltpu.sync_copy(tail_hbm, pk_v.at[0, pl.ds(0, nt)])
                pltpu.sync_copy(
                    pk_v.at[0, pl.ds(0, nt)],
                    slabs_hbm.at[pl.ds(n_units * Q, nt)],
                )

    return k


def _gather_kernel(B, D, b_per_w, CH, NC):
    n_ch = b_per_w // CH
    SW = D * _G
    mesh = plsc.VectorSubcoreMesh(core_axis_name="c", subcore_axis_name="s")

    @functools.partial(
        pl.kernel,
        mesh=mesh,
        out_type=jax.ShapeDtypeStruct((D, B), jnp.float32),
        scratch_types=[
            pltpu.VMEM((b_per_w,), jnp.int32),
            pltpu.VMEM((b_per_w,), jnp.int32),
            pltpu.VMEM((2, CH, SW), jnp.float32),
            pltpu.VMEM((D, b_per_w), jnp.float32),
            pltpu.SemaphoreType.DMA,
        ],
        compiler_params=_SC_PARAMS,
    )
    def k(slabs_hbm, idx_hbm, outT_hbm, idx_v, blk_v, rows_v, cols_v, sem):
        wid = lax.axis_index("s") * NC + lax.axis_index("c")
        base = wid * b_per_w
        pltpu.sync_copy(idx_hbm.at[pl.ds(base, b_per_w)], idx_v)
        for v in range(b_per_w // 16):
            iv = idx_v[pl.ds(v * 16, 16)]
            blk_v[pl.ds(v * 16, 16)] = iv >> 4

        def start(c, b):
            @pl.when(c < n_ch)
            def _():
                i0 = pl.multiple_of(c * CH, CH)
                pltpu.async_copy(
                    slabs_hbm.at[blk_v.at[pl.ds(i0, CH)]],
                    rows_v.at[b],
                    sem,
                )

        def drain():
            pltpu.make_async_copy(
                slabs_hbm.at[blk_v.at[pl.ds(0, CH)]], rows_v.at[0], sem
            ).wait()

        lanes = lax.iota(jnp.int32, 16)
        start(0, 0)

        def body(c2, _):
            for b in range(2):
                c = c2 * 2 + b
                drain()
                start(c + 1, 1 - b)
                buf = rows_v.at[b]
                for g in range(CH // 16):
                    j = c * CH + g * 16
                    iv = idx_v[pl.ds(j, 16)]
                    sub = iv & (_G - 1)
                    rowids = lanes + (g * 16)
                    for col in range(D):
                        vals = plsc.load_gather(
                            buf, [rowids, sub + (col * _G)]
                        )
                        cols_v[col, pl.ds(j, 16)] = vals
            return 0

        lax.fori_loop(0, n_ch // 2, body, 0)
        pltpu.sync_copy(cols_v, outT_hbm.at[:, pl.ds(base, b_per_w)])

    return k


def kernel(input, index):
    V, D = input.shape
    B = index.shape[0]
    idx32 = index.astype(jnp.int32)
    tableT = input.T
    tail = V - (V // 256) * 256
    tail_packed = (
        input[V - tail :, :].reshape(tail // _G, _G, D)
        .transpose(0, 2, 1)
        .reshape(tail // _G, D * _G)
    )

    info = plsc.get_sparse_core_info()
    NC, NS = info.num_cores, info.num_subcores
    NW = NC * NS
    b_per_w = B // NW

    slabs = _relayout_kernel(V, D, NC, NW)(tableT, tail_packed)
    outT = _gather_kernel(B, D, b_per_w, 32, NC)(slabs, idx32)
    return outT.T


# speedup vs baseline: 5.5972x; 1.5755x over previous
import functools

import jax
import jax.numpy as jnp
from jax import lax
from jax.experimental import pallas as pl
from jax.experimental.pallas import tpu as pltpu
from jax.experimental.pallas import tpu_sc as plsc

_SC_PARAMS = pltpu.CompilerParams(
    use_tc_tiling_on_sc=True, needs_layout_passes=False
)
_G = 16


def _relayout_kernel(V, D, NC, NW):
    U = 256
    Q = U // _G
    SW = D * _G
    n_units = V // U
    tail = V - n_units * U
    n_slabs = (V + _G - 1) // _G
    iters = (n_units + NW - 1) // NW
    iters += iters % 2
    mesh = plsc.VectorSubcoreMesh(core_axis_name="c", subcore_axis_name="s")

    @functools.partial(
        pl.kernel,
        mesh=mesh,
        out_type=jax.ShapeDtypeStruct((n_slabs, SW), jnp.float32),
        scratch_types=[
            pltpu.VMEM((2, D, U), jnp.float32),
            pltpu.VMEM((2, Q, SW), jnp.float32),
            pltpu.SemaphoreType.DMA,
            pltpu.SemaphoreType.DMA,
        ],
        compiler_params=_SC_PARAMS,
    )
    def k(tableT_hbm, tail_hbm, slabs_hbm, nat_v, pk_v, sin, sout):
        wid = lax.axis_index("s") * NC + lax.axis_index("c")

        def unit_id(i):
            return wid + i * NW

        def stage(i, b):
            u = unit_id(i)

            @pl.when(u < n_units)
            def _():
                lane0 = pl.multiple_of(u * U, U)
                pltpu.async_copy(
                    tableT_hbm.at[:, pl.ds(lane0, U)], nat_v.at[b], sin
                )

        def drain_stage(i, b):
            @pl.when(unit_id(i) < n_units)
            def _():
                pltpu.make_async_copy(
                    tableT_hbm.at[:, pl.ds(0, U)], nat_v.at[b], sin
                ).wait()

        def write(i, b):
            u = unit_id(i)

            @pl.when(u < n_units)
            def _():
                row0 = pl.multiple_of(u * Q, Q)
                pltpu.async_copy(
                    pk_v.at[b], slabs_hbm.at[pl.ds(row0, Q)], sout
                )

        def drain_write(i, b):
            @pl.when(jnp.logical_and(i >= 0, unit_id(i) < n_units))
            def _():
                pltpu.make_async_copy(
                    pk_v.at[b], slabs_hbm.at[pl.ds(0, Q)], sout
                ).wait()

        def shuffle(b):
            nat = nat_v.at[b]
            pk = pk_v.at[b]

            @plsc.parallel_loop(0, Q, 1, unroll=8)
            def row(q):
                l0 = q * _G
                for c in range(D):
                    pk[q, pl.ds(c * _G, _G)] = nat[c, pl.ds(l0, _G)]

        stage(0, 0)
        def body(i2_, _):
            i2 = i2_ * 2
            for b in range(2):
                i = i2 + b
                u = unit_id(i)
                drain_stage(i, b)
                stage(i + 1, 1 - b)
                drain_write(i - 2, b)

                @pl.when(u < n_units)
                def _():
                    shuffle(b)

                write(i, b)
            return 0

        lax.fori_loop(0, iters // 2, body, 0)
        drain_stage(iters, iters % 2)
        drain_write(iters - 2, 0)
        drain_write(iters - 1, 1)

        if tail:
            @pl.when(wid == 0)
            def _():
                nt = tail // _G
                pltpu.sync_copy(tail_hbm, pk_v.at[0, pl.ds(0, nt)])
                pltpu.sync_copy(
                    pk_v.at[0, pl.ds(0, nt)],
                    slabs_hbm.at[pl.ds(n_units * Q, nt)],
                )

    return k


def _gather_kernel(B, D, b_per_w, CH, NC):
    n_ch = b_per_w // CH
    SW = D * _G
    mesh = plsc.VectorSubcoreMesh(core_axis_name="c", subcore_axis_name="s")

    @functools.partial(
        pl.kernel,
        mesh=mesh,
        out_type=jax.ShapeDtypeStruct((D, B), jnp.float32),
        scratch_types=[
            pltpu.VMEM((b_per_w,), jnp.int32),
            pltpu.VMEM((b_per_w,), jnp.int32),
            pltpu.VMEM((2, CH, SW), jnp.float32),
            pltpu.VMEM((D, b_per_w), jnp.float32),
            pltpu.SemaphoreType.DMA,
        ],
        compiler_params=_SC_PARAMS,
    )
    def k(slabs_hbm, idx_hbm, outT_hbm, idx_v, blk_v, rows_v, cols_v, sem):
        wid = lax.axis_index("s") * NC + lax.axis_index("c")
        base = wid * b_per_w
        pltpu.sync_copy(idx_hbm.at[pl.ds(base, b_per_w)], idx_v)
        for v in range(b_per_w // 16):
            iv = idx_v[pl.ds(v * 16, 16)]
            blk_v[pl.ds(v * 16, 16)] = iv >> 4

        def start(c, b):
            @pl.when(c < n_ch)
            def _():
                i0 = pl.multiple_of(c * CH, CH)
                pltpu.async_copy(
                    slabs_hbm.at[blk_v.at[pl.ds(i0, CH)]],
                    rows_v.at[b],
                    sem,
                )

        def drain():
            pltpu.make_async_copy(
                slabs_hbm.at[blk_v.at[pl.ds(0, CH)]], rows_v.at[0], sem
            ).wait()

        lanes = lax.iota(jnp.int32, 16)
        start(0, 0)

        def body(c2, _):
            for b in range(2):
                c = c2 * 2 + b
                drain()
                start(c + 1, 1 - b)
                buf = rows_v.at[b]
                for g in range(CH // 16):
                    j = c * CH + g * 16
                    iv = idx_v[pl.ds(j, 16)]
                    sub = iv & (_G - 1)
                    rowids = lanes + (g * 16)
                    for col in range(D):
                        vals = plsc.load_gather(
                            buf, [rowids, sub + (col * _G)]
                        )
                        cols_v[col, pl.ds(j, 16)] = vals
            return 0

        lax.fori_loop(0, n_ch // 2, body, 0)
        pltpu.sync_copy(cols_v, outT_hbm.at[:, pl.ds(base, b_per_w)])

    return k


def kernel(input, index):
    V, D = input.shape
    B = index.shape[0]
    idx32 = index.astype(jnp.int32)
    tableT = input.T
    tail = V - (V // 256) * 256
    tail_packed = (
        input[V - tail :, :].reshape(tail // _G, _G, D)
        .transpose(0, 2, 1)
        .reshape(tail // _G, D * _G)
    )

    info = plsc.get_sparse_core_info()
    NC, NS = info.num_cores, info.num_subcores
    NW = NC * NS
    b_per_w = B // NW

    slabs = _relayout_kernel(V, D, NC, NW)(tableT, tail_packed)
    outT = _gather_kernel(B, D, b_per_w, 32, NC)(slabs, idx32)
    return outT.T
